# baseline (speedup 1.0000x reference)
import jax
import jax.numpy as jnp
from jax import lax
from jax.experimental import pallas as pl
from jax.experimental.pallas import tpu as pltpu
from jax.experimental.pallas import tpu_sc as plsc

NC = 2
NS = 16
NW = NC * NS
SEQ = 50
EMB = 64
ROWS = 128
NBUF = 4
LANES = 16
RBLK = 512


def _proj_body(t_ref, w_ref, b_ref, o_ref):
    p = lax.dot_general(w_ref[...], t_ref[...], (((1,), (0,)), ((), ())),
                        preferred_element_type=jnp.float32)
    p = p + b_ref[...]
    pu = lax.bitcast_convert_type(p.astype(jnp.bfloat16), jnp.uint16)
    pu = pu.astype(jnp.int32)
    word = (pu[1:2, :] << 16) | pu[0:1, :]
    o_ref[...] = word.reshape(RBLK, 128)


def _sc_body(x_hbm, pt_hbm, out_hbm, idx_v, g0, g1, g2, g3,
             acc0, acc1, pt_s, s0, s1, s2, s3):
    cid = lax.axis_index("c")
    tid = lax.axis_index("s")
    wid = cid * NS + tid
    b_per_w = acc0.shape[0]
    base = wid * b_per_w
    nj = b_per_w // ROWS
    bufs = (g0, g1, g2, g3)
    sems = (s0, s1, s2, s3)
    n_rows = pt_s.shape[0]

    sl = n_rows // 8

    @pl.when(tid < 8)
    def _():
        pltpu.sync_copy(pt_hbm.at[pl.ds(tid * sl, sl)],
                        pt_s.at[pl.ds(tid * sl, sl)])

    pltpu.sync_copy(x_hbm.at[:, pl.ds(base, b_per_w)], idx_v)

    zero = jnp.zeros((LANES,), jnp.float32)
    for g in range(b_per_w // LANES):
        acc0[pl.ds(g * LANES, LANES)] = zero
        acc1[pl.ds(g * LANES, LANES)] = zero

    ngroups = b_per_w // LANES

    plsc.subcore_barrier()

    def _gather(s, j, buf, sem):
        return pltpu.make_async_copy(
            pt_hbm.at[idx_v.at[s, pl.ds(j * ROWS, ROWS)]], buf, sem)

    for j in range(nj):
        _gather(0, j, bufs[j], sems[j]).start()

    himask = jnp.int32(-65536)

    def _step(s, carry):
        for j in range(nj):
            _gather(s, j, bufs[j], sems[j]).wait()
            for g in range(ROWS // LANES):
                off = j * ROWS + g * LANES
                v = bufs[j][pl.ds(off - j * ROWS, LANES)]
                a = lax.bitcast_convert_type(v << 16, jnp.float32)
                b = lax.bitcast_convert_type(v & himask, jnp.float32)
                plsc.addupdate(acc0.at[pl.ds(off, LANES)], a)
                plsc.addupdate(acc1.at[pl.ds(off, LANES)], b)

            @pl.when(s + 1 < SEQ)
            def _():
                _gather(s + 1, j, bufs[j], sems[j]).start()
        return carry

    lax.fori_loop(0, SEQ, _step, 0)

    inv = jnp.float32(1.0 / SEQ)
    for g in range(b_per_w // LANES):
        s_ = pl.ds(g * LANES, LANES)
        acc0[s_] = acc0[s_] * inv
        acc1[s_] = acc1[s_] * inv

    pltpu.sync_copy(acc0, out_hbm.at[0, pl.ds(base, b_per_w)])
    pltpu.sync_copy(acc1, out_hbm.at[1, pl.ds(base, b_per_w)])


def kernel(x, table, W, b):
    seq, batch = x.shape
    vocab = table.shape[0]
    b_per_w = batch // NW
    ptrows = (vocab + 127) // 128
    pt_n = ptrows * 128
    grid = (ptrows + RBLK - 1) // RBLK

    tbl_t = table.T

    pt2d = pl.pallas_call(
        _proj_body,
        grid=(grid,),
        in_specs=[
            pl.BlockSpec((EMB, RBLK * 128), lambda i: (0, i)),
            pl.BlockSpec((2, EMB), lambda i: (0, 0)),
            pl.BlockSpec((2, 1), lambda i: (0, 0)),
        ],
        out_specs=pl.BlockSpec((RBLK, 128), lambda i: (i, 0)),
        out_shape=jax.ShapeDtypeStruct((ptrows, 128), jnp.int32),
    )(tbl_t, W, b.reshape(2, 1))
    pt16 = pt2d.reshape(pt_n)

    sc = pl.kernel(
        _sc_body,
        out_type=jax.ShapeDtypeStruct((2, batch), jnp.float32),
        mesh=plsc.VectorSubcoreMesh(core_axis_name="c", subcore_axis_name="s"),
        compiler_params=pltpu.CompilerParams(use_tc_tiling_on_sc=False),
        scratch_types=[
            pltpu.VMEM((seq, b_per_w), jnp.int32),
            pltpu.VMEM((ROWS,), jnp.int32),
            pltpu.VMEM((ROWS,), jnp.int32),
            pltpu.VMEM((ROWS,), jnp.int32),
            pltpu.VMEM((ROWS,), jnp.int32),
            pltpu.VMEM((b_per_w,), jnp.float32),
            pltpu.VMEM((b_per_w,), jnp.float32),
            pltpu.VMEM_SHARED((pt_n,), jnp.int32),
            pltpu.SemaphoreType.DMA,
            pltpu.SemaphoreType.DMA,
            pltpu.SemaphoreType.DMA,
            pltpu.SemaphoreType.DMA,
        ],
    )
    out2 = sc(x.astype(jnp.int32), pt16)
    return out2.T


# speedup vs baseline: 1.0147x; 1.0147x over previous
import jax
import jax.numpy as jnp
from jax import lax
from jax.experimental import pallas as pl
from jax.experimental.pallas import tpu as pltpu
from jax.experimental.pallas import tpu_sc as plsc

NC = 2
NS = 16
NW = NC * NS
SEQ = 50
EMB = 64
ROWS = 128
NBUF = 4
LANES = 16
RBLK = 256


def _proj_body(t_ref, w_ref, b_ref, o_ref):
    p = lax.dot_general(w_ref[...], t_ref[...], (((1,), (0,)), ((), ())),
                        preferred_element_type=jnp.float32)
    p = p + b_ref[...]
    pu = lax.bitcast_convert_type(p.astype(jnp.bfloat16), jnp.uint16)
    pu = pu.astype(jnp.int32)
    word = (pu[1:2, :] << 16) | pu[0:1, :]
    o_ref[...] = word.reshape(RBLK, 128)


def _sc_body(x_hbm, pt_hbm, out_hbm, idx_v, g0, g1, g2, g3,
             acc0, acc1, pt_s, s0, s1, s2, s3):
    cid = lax.axis_index("c")
    tid = lax.axis_index("s")
    wid = cid * NS + tid
    b_per_w = acc0.shape[0]
    base = wid * b_per_w
    nj = b_per_w // ROWS
    bufs = (g0, g1, g2, g3)
    sems = (s0, s1, s2, s3)
    n_rows = pt_s.shape[0]

    sl = n_rows // 8

    @pl.when(tid < 8)
    def _():
        pltpu.sync_copy(pt_hbm.at[pl.ds(tid * sl, sl)],
                        pt_s.at[pl.ds(tid * sl, sl)])

    pltpu.sync_copy(x_hbm.at[:, pl.ds(base, b_per_w)], idx_v)

    zero = jnp.zeros((LANES,), jnp.float32)
    for g in range(b_per_w // LANES):
        acc0[pl.ds(g * LANES, LANES)] = zero
        acc1[pl.ds(g * LANES, LANES)] = zero

    ngroups = b_per_w // LANES

    plsc.subcore_barrier()

    def _gather(s, j, buf, sem):
        return pltpu.make_async_copy(
            pt_hbm.at[idx_v.at[s, pl.ds(j * ROWS, ROWS)]], buf, sem)

    for j in range(nj):
        _gather(0, j, bufs[j], sems[j]).start()

    himask = jnp.int32(-65536)

    def _step(s, carry):
        for j in range(nj):
            _gather(s, j, bufs[j], sems[j]).wait()
            for g in range(ROWS // LANES):
                off = j * ROWS + g * LANES
                v = bufs[j][pl.ds(off - j * ROWS, LANES)]
                a = lax.bitcast_convert_type(v << 16, jnp.float32)
                b = lax.bitcast_convert_type(v & himask, jnp.float32)
                plsc.addupdate(acc0.at[pl.ds(off, LANES)], a)
                plsc.addupdate(acc1.at[pl.ds(off, LANES)], b)

            @pl.when(s + 1 < SEQ)
            def _():
                _gather(s + 1, j, bufs[j], sems[j]).start()
        return carry

    lax.fori_loop(0, SEQ, _step, 0)

    inv = jnp.float32(1.0 / SEQ)
    for g in range(b_per_w // LANES):
        s_ = pl.ds(g * LANES, LANES)
        acc0[s_] = acc0[s_] * inv
        acc1[s_] = acc1[s_] * inv

    pltpu.sync_copy(acc0, out_hbm.at[0, pl.ds(base, b_per_w)])
    pltpu.sync_copy(acc1, out_hbm.at[1, pl.ds(base, b_per_w)])


def kernel(x, table, W, b):
    seq, batch = x.shape
    vocab = table.shape[0]
    b_per_w = batch // NW
    ptrows = (vocab + 127) // 128
    pt_n = ptrows * 128
    grid = (ptrows + RBLK - 1) // RBLK

    tbl_t = table.T

    pt2d = pl.pallas_call(
        _proj_body,
        grid=(grid,),
        in_specs=[
            pl.BlockSpec((EMB, RBLK * 128), lambda i: (0, i)),
            pl.BlockSpec((2, EMB), lambda i: (0, 0)),
            pl.BlockSpec((2, 1), lambda i: (0, 0)),
        ],
        out_specs=pl.BlockSpec((RBLK, 128), lambda i: (i, 0)),
        out_shape=jax.ShapeDtypeStruct((ptrows, 128), jnp.int32),
    )(tbl_t, W, b.reshape(2, 1))
    pt16 = pt2d.reshape(pt_n)

    sc = pl.kernel(
        _sc_body,
        out_type=jax.ShapeDtypeStruct((2, batch), jnp.float32),
        mesh=plsc.VectorSubcoreMesh(core_axis_name="c", subcore_axis_name="s"),
        compiler_params=pltpu.CompilerParams(use_tc_tiling_on_sc=False),
        scratch_types=[
            pltpu.VMEM((seq, b_per_w), jnp.int32),
            pltpu.VMEM((ROWS,), jnp.int32),
            pltpu.VMEM((ROWS,), jnp.int32),
            pltpu.VMEM((ROWS,), jnp.int32),
            pltpu.VMEM((ROWS,), jnp.int32),
            pltpu.VMEM((b_per_w,), jnp.float32),
            pltpu.VMEM((b_per_w,), jnp.float32),
            pltpu.VMEM_SHARED((pt_n,), jnp.int32),
            pltpu.SemaphoreType.DMA,
            pltpu.SemaphoreType.DMA,
            pltpu.SemaphoreType.DMA,
            pltpu.SemaphoreType.DMA,
        ],
    )
    out2 = sc(x.astype(jnp.int32), pt16)
    return out2.T


# speedup vs baseline: 1.3352x; 1.3159x over previous
import jax
import jax.numpy as jnp
from jax import lax
from jax.experimental import pallas as pl
from jax.experimental.pallas import tpu as pltpu
from jax.experimental.pallas import tpu_sc as plsc

NC = 2
NS = 16
NW = NC * NS
SEQ = 50
EMB = 64
ROWS = 128
NBUF = 4
LANES = 16
RBLK = 256


def _proj_body(t_ref, w_ref, b_ref, o_ref):
    p = lax.dot_general(w_ref[...], t_ref[...], (((1,), (0,)), ((), ())),
                        preferred_element_type=jnp.float32)
    p = p + b_ref[...]
    pu = lax.bitcast_convert_type(p.astype(jnp.bfloat16), jnp.uint16)
    pu = pu.astype(jnp.int32)
    word = (pu[1:2, :] << 16) | pu[0:1, :]
    o_ref[...] = word.reshape(RBLK, 128)


def _sc_body(x_hbm, pt_hbm, out_hbm, idx_v, g0, g1, g2, g3,
             acc0, acc1, pt_s, s0, s1, s2, s3):
    cid = lax.axis_index("c")
    tid = lax.axis_index("s")
    wid = cid * NS + tid
    b_per_w = acc0.shape[0]
    base = wid * b_per_w
    nj = b_per_w // ROWS
    bufs = (g0, g1, g2, g3)
    sems = (s0, s1, s2, s3)
    n_rows = pt_s.shape[0]

    sl = n_rows // 8

    @pl.when(tid == 0)
    def _():
        pltpu.sync_copy(pt_hbm, pt_s)

    pltpu.sync_copy(x_hbm.at[:, pl.ds(base, b_per_w)], idx_v)

    zero = jnp.zeros((LANES,), jnp.float32)
    for g in range(b_per_w // LANES):
        acc0[pl.ds(g * LANES, LANES)] = zero
        acc1[pl.ds(g * LANES, LANES)] = zero

    ngroups = b_per_w // LANES

    plsc.subcore_barrier()

    def _gather(s, j, buf, sem):
        return pltpu.make_async_copy(
            pt_s.at[idx_v.at[s, pl.ds(j * ROWS, ROWS)]], buf, sem)

    for j in range(nj):
        _gather(0, j, bufs[j], sems[j]).start()

    himask = jnp.int32(-65536)

    def _step(s, carry):
        for j in range(nj):
            _gather(s, j, bufs[j], sems[j]).wait()
            for g in range(ROWS // LANES):
                off = j * ROWS + g * LANES
                v = bufs[j][pl.ds(off - j * ROWS, LANES)]
                a = lax.bitcast_convert_type(v << 16, jnp.float32)
                b = lax.bitcast_convert_type(v & himask, jnp.float32)
                plsc.addupdate(acc0.at[pl.ds(off, LANES)], a)
                plsc.addupdate(acc1.at[pl.ds(off, LANES)], b)

            @pl.when(s + 1 < SEQ)
            def _():
                _gather(s + 1, j, bufs[j], sems[j]).start()
        return carry

    lax.fori_loop(0, SEQ, _step, 0)

    inv = jnp.float32(1.0 / SEQ)
    for g in range(b_per_w // LANES):
        s_ = pl.ds(g * LANES, LANES)
        acc0[s_] = acc0[s_] * inv
        acc1[s_] = acc1[s_] * inv

    pltpu.sync_copy(acc0, out_hbm.at[0, pl.ds(base, b_per_w)])
    pltpu.sync_copy(acc1, out_hbm.at[1, pl.ds(base, b_per_w)])


def kernel(x, table, W, b):
    seq, batch = x.shape
    vocab = table.shape[0]
    b_per_w = batch // NW
    ptrows = (vocab + 127) // 128
    pt_n = ptrows * 128
    grid = (ptrows + RBLK - 1) // RBLK

    tbl_t = table.T

    pt2d = pl.pallas_call(
        _proj_body,
        grid=(grid,),
        in_specs=[
            pl.BlockSpec((EMB, RBLK * 128), lambda i: (0, i)),
            pl.BlockSpec((2, EMB), lambda i: (0, 0)),
            pl.BlockSpec((2, 1), lambda i: (0, 0)),
        ],
        out_specs=pl.BlockSpec((RBLK, 128), lambda i: (i, 0)),
        out_shape=jax.ShapeDtypeStruct((ptrows, 128), jnp.int32),
    )(tbl_t, W, b.reshape(2, 1))
    pt16 = pt2d.reshape(pt_n)

    sc = pl.kernel(
        _sc_body,
        out_type=jax.ShapeDtypeStruct((2, batch), jnp.float32),
        mesh=plsc.VectorSubcoreMesh(core_axis_name="c", subcore_axis_name="s"),
        compiler_params=pltpu.CompilerParams(use_tc_tiling_on_sc=False),
        scratch_types=[
            pltpu.VMEM((seq, b_per_w), jnp.int32),
            pltpu.VMEM((ROWS,), jnp.int32),
            pltpu.VMEM((ROWS,), jnp.int32),
            pltpu.VMEM((ROWS,), jnp.int32),
            pltpu.VMEM((ROWS,), jnp.int32),
            pltpu.VMEM((b_per_w,), jnp.float32),
            pltpu.VMEM((b_per_w,), jnp.float32),
            pltpu.VMEM_SHARED((pt_n,), jnp.int32),
            pltpu.SemaphoreType.DMA,
            pltpu.SemaphoreType.DMA,
            pltpu.SemaphoreType.DMA,
            pltpu.SemaphoreType.DMA,
        ],
    )
    out2 = sc(x.astype(jnp.int32), pt16)
    return out2.T


# speedup vs baseline: 1.3377x; 1.0019x over previous
import jax
import jax.numpy as jnp
from jax import lax
from jax.experimental import pallas as pl
from jax.experimental.pallas import tpu as pltpu
from jax.experimental.pallas import tpu_sc as plsc

NC = 2
NS = 16
NW = NC * NS
SEQ = 50
EMB = 64
ROWS = 128
NBUF = 4
LANES = 16
RBLK = 256


def _proj_body(t_ref, w_ref, b_ref, o_ref):
    p = lax.dot_general(w_ref[...], t_ref[...], (((1,), (0,)), ((), ())),
                        preferred_element_type=jnp.float32)
    p = p + b_ref[...]
    pu = lax.bitcast_convert_type(p.astype(jnp.bfloat16), jnp.uint16)
    pu = pu.astype(jnp.int32)
    word = (pu[1:2, :] << 16) | pu[0:1, :]
    o_ref[...] = word.reshape(RBLK, 128)


def _sc_body(x_hbm, pt_hbm, out_hbm, idx_v, g0, g1, g2, g3,
             acc0, acc1, pt_s, s0, s1, s2, s3):
    cid = lax.axis_index("c")
    tid = lax.axis_index("s")
    wid = cid * NS + tid
    b_per_w = acc0.shape[0]
    base = wid * b_per_w
    nj = b_per_w // ROWS
    bufs = (g0, g1, g2, g3)
    sems = (s0, s1, s2, s3)
    n_rows = pt_s.shape[0]

    sl = n_rows // 8

    chunk = 489 * 128
    tail = n_rows - 15 * chunk

    @pl.when(tid < 15)
    def _():
        pltpu.sync_copy(pt_hbm.at[pl.ds(tid * chunk, chunk)],
                        pt_s.at[pl.ds(tid * chunk, chunk)])

    @pl.when(tid == 15)
    def _():
        pltpu.sync_copy(pt_hbm.at[pl.ds(15 * chunk, tail)],
                        pt_s.at[pl.ds(15 * chunk, tail)])

    pltpu.sync_copy(x_hbm.at[:, pl.ds(base, b_per_w)], idx_v)

    zero = jnp.zeros((LANES,), jnp.float32)
    for g in range(b_per_w // LANES):
        acc0[pl.ds(g * LANES, LANES)] = zero
        acc1[pl.ds(g * LANES, LANES)] = zero

    ngroups = b_per_w // LANES

    plsc.subcore_barrier()

    def _gather(s, j, buf, sem):
        return pltpu.make_async_copy(
            pt_s.at[idx_v.at[s, pl.ds(j * ROWS, ROWS)]], buf, sem)

    for j in range(nj):
        _gather(0, j, bufs[j], sems[j]).start()

    himask = jnp.int32(-65536)

    def _step(s, carry):
        for j in range(nj):
            _gather(s, j, bufs[j], sems[j]).wait()
            for g in range(ROWS // LANES):
                off = j * ROWS + g * LANES
                v = bufs[j][pl.ds(off - j * ROWS, LANES)]
                a = lax.bitcast_convert_type(v << 16, jnp.float32)
                b = lax.bitcast_convert_type(v & himask, jnp.float32)
                plsc.addupdate(acc0.at[pl.ds(off, LANES)], a)
                plsc.addupdate(acc1.at[pl.ds(off, LANES)], b)

            @pl.when(s + 1 < SEQ)
            def _():
                _gather(s + 1, j, bufs[j], sems[j]).start()
        return carry

    lax.fori_loop(0, SEQ, _step, 0)

    inv = jnp.float32(1.0 / SEQ)
    for g in range(b_per_w // LANES):
        s_ = pl.ds(g * LANES, LANES)
        acc0[s_] = acc0[s_] * inv
        acc1[s_] = acc1[s_] * inv

    pltpu.sync_copy(acc0, out_hbm.at[0, pl.ds(base, b_per_w)])
    pltpu.sync_copy(acc1, out_hbm.at[1, pl.ds(base, b_per_w)])


def kernel(x, table, W, b):
    seq, batch = x.shape
    vocab = table.shape[0]
    b_per_w = batch // NW
    ptrows = (vocab + 127) // 128
    pt_n = ptrows * 128
    grid = (ptrows + RBLK - 1) // RBLK

    tbl_t = table.T

    pt2d = pl.pallas_call(
        _proj_body,
        grid=(grid,),
        in_specs=[
            pl.BlockSpec((EMB, RBLK * 128), lambda i: (0, i)),
            pl.BlockSpec((2, EMB), lambda i: (0, 0)),
            pl.BlockSpec((2, 1), lambda i: (0, 0)),
        ],
        out_specs=pl.BlockSpec((RBLK, 128), lambda i: (i, 0)),
        out_shape=jax.ShapeDtypeStruct((ptrows, 128), jnp.int32),
    )(tbl_t, W, b.reshape(2, 1))
    pt16 = pt2d.reshape(pt_n)

    sc = pl.kernel(
        _sc_body,
        out_type=jax.ShapeDtypeStruct((2, batch), jnp.float32),
        mesh=plsc.VectorSubcoreMesh(core_axis_name="c", subcore_axis_name="s"),
        compiler_params=pltpu.CompilerParams(use_tc_tiling_on_sc=False),
        scratch_types=[
            pltpu.VMEM((seq, b_per_w), jnp.int32),
            pltpu.VMEM((ROWS,), jnp.int32),
            pltpu.VMEM((ROWS,), jnp.int32),
            pltpu.VMEM((ROWS,), jnp.int32),
            pltpu.VMEM((ROWS,), jnp.int32),
            pltpu.VMEM((b_per_w,), jnp.float32),
            pltpu.VMEM((b_per_w,), jnp.float32),
            pltpu.VMEM_SHARED((pt_n,), jnp.int32),
            pltpu.SemaphoreType.DMA,
            pltpu.SemaphoreType.DMA,
            pltpu.SemaphoreType.DMA,
            pltpu.SemaphoreType.DMA,
        ],
    )
    out2 = sc(x.astype(jnp.int32), pt16)
    return out2.T


# speedup vs baseline: 1.3378x; 1.0001x over previous
import jax
import jax.numpy as jnp
from jax import lax
from jax.experimental import pallas as pl
from jax.experimental.pallas import tpu as pltpu
from jax.experimental.pallas import tpu_sc as plsc

NC = 2
NS = 16
NW = NC * NS
SEQ = 50
EMB = 64
ROWS = 128
LANES = 16
RBLK = 256


def _proj_body(t_ref, w_ref, b_ref, o_ref):
    p = lax.dot_general(w_ref[...], t_ref[...], (((1,), (0,)), ((), ())),
                        preferred_element_type=jnp.float32)
    p = p + b_ref[...]
    pu = lax.bitcast_convert_type(p.astype(jnp.bfloat16), jnp.uint16)
    pu = pu.astype(jnp.int32)
    word = (pu[1:2, :] << 16) | pu[0:1, :]
    o_ref[...] = word.reshape(RBLK, 128)


def _sc_body(x_hbm, pt_hbm, out_hbm, idx_v, g0, g1, g2, g3,
             acc0, acc1, pt_s, s0, s1, s2, s3):
    cid = lax.axis_index("c")
    tid = lax.axis_index("s")
    wid = cid * NS + tid
    b_per_w = acc0.shape[0]
    base = wid * b_per_w
    nj = b_per_w // ROWS
    bufs = (g0, g1, g2, g3)
    sems = (s0, s1, s2, s3)
    n_rows = pt_s.shape[0]

    sl = n_rows // 8

    chunk = 489 * 128
    tail = n_rows - 15 * chunk

    @pl.when(tid < 15)
    def _():
        pltpu.sync_copy(pt_hbm.at[pl.ds(tid * chunk, chunk)],
                        pt_s.at[pl.ds(tid * chunk, chunk)])

    @pl.when(tid == 15)
    def _():
        pltpu.sync_copy(pt_hbm.at[pl.ds(15 * chunk, tail)],
                        pt_s.at[pl.ds(15 * chunk, tail)])

    pltpu.sync_copy(x_hbm.at[:, pl.ds(base, b_per_w)], idx_v)

    zero = jnp.zeros((LANES,), jnp.float32)
    for g in range(b_per_w // LANES):
        acc0[pl.ds(g * LANES, LANES)] = zero
        acc1[pl.ds(g * LANES, LANES)] = zero

    plsc.subcore_barrier()

    def _gather(s, j, buf, sem):
        return pltpu.make_async_copy(
            pt_s.at[idx_v.at[s, pl.ds(j * ROWS, ROWS)]], buf, sem)

    for j in range(nj):
        _gather(0, j, bufs[j], sems[j]).start()

    himask = jnp.int32(-65536)

    def _step(s, carry):
        for j in range(nj):
            _gather(s, j, bufs[j], sems[j]).wait()
            for g in range(ROWS // LANES):
                off = j * ROWS + g * LANES
                v = bufs[j][pl.ds(off - j * ROWS, LANES)]
                a = lax.bitcast_convert_type(v << 16, jnp.float32)
                b = lax.bitcast_convert_type(v & himask, jnp.float32)
                plsc.addupdate(acc0.at[pl.ds(off, LANES)], a)
                plsc.addupdate(acc1.at[pl.ds(off, LANES)], b)

            @pl.when(s + 1 < SEQ)
            def _():
                _gather(s + 1, j, bufs[j], sems[j]).start()
        return carry

    lax.fori_loop(0, SEQ, _step, 0)

    inv = jnp.float32(1.0 / SEQ)
    for g in range(b_per_w // LANES):
        s_ = pl.ds(g * LANES, LANES)
        acc0[s_] = acc0[s_] * inv
        acc1[s_] = acc1[s_] * inv

    pltpu.sync_copy(acc0, out_hbm.at[0, pl.ds(base, b_per_w)])
    pltpu.sync_copy(acc1, out_hbm.at[1, pl.ds(base, b_per_w)])


def kernel(x, table, W, b):
    seq, batch = x.shape
    vocab = table.shape[0]
    b_per_w = batch // NW
    ptrows = (vocab + 127) // 128
    pt_n = ptrows * 128
    grid = (ptrows + RBLK - 1) // RBLK

    tbl_t = table.T

    pt2d = pl.pallas_call(
        _proj_body,
        grid=(grid,),
        in_specs=[
            pl.BlockSpec((EMB, RBLK * 128), lambda i: (0, i)),
            pl.BlockSpec((2, EMB), lambda i: (0, 0)),
            pl.BlockSpec((2, 1), lambda i: (0, 0)),
        ],
        out_specs=pl.BlockSpec((RBLK, 128), lambda i: (i, 0)),
        out_shape=jax.ShapeDtypeStruct((ptrows, 128), jnp.int32),
    )(tbl_t, W, b.reshape(2, 1))
    pt16 = pt2d.reshape(pt_n)

    sc = pl.kernel(
        _sc_body,
        out_type=jax.ShapeDtypeStruct((2, batch), jnp.float32),
        mesh=plsc.VectorSubcoreMesh(core_axis_name="c", subcore_axis_name="s"),
        compiler_params=pltpu.CompilerParams(use_tc_tiling_on_sc=False),
        scratch_types=[
            pltpu.VMEM((seq, b_per_w), jnp.int32),
            pltpu.VMEM((ROWS,), jnp.int32),
            pltpu.VMEM((ROWS,), jnp.int32),
            pltpu.VMEM((ROWS,), jnp.int32),
            pltpu.VMEM((ROWS,), jnp.int32),
            pltpu.VMEM((b_per_w,), jnp.float32),
            pltpu.VMEM((b_per_w,), jnp.float32),
            pltpu.VMEM_SHARED((pt_n,), jnp.int32),
            pltpu.SemaphoreType.DMA,
            pltpu.SemaphoreType.DMA,
            pltpu.SemaphoreType.DMA,
            pltpu.SemaphoreType.DMA,
        ],
    )
    out2 = sc(x.astype(jnp.int32), pt16)
    return out2.T
